# baseline (speedup 1.0000x reference)
import functools
import math

import jax
import jax.numpy as jnp
import numpy as np
from jax import lax
from jax.experimental import pallas as pl
from jax.experimental.pallas import tpu as pltpu, tpu_sc as plsc

NBINS = 9
H = W = 224
HP = WP = 28
NCELL = HP * WP
FEAT = NCELL * NBINS
NTILES = 32
IMGS_PER_TILE = 64 // NTILES

_COS = [math.cos(k * math.pi / NBINS) for k in range(1, NBINS)]
_SIN = [math.sin(k * math.pi / NBINS) for k in range(1, NBINS)]


def _bf16_round(v):
    bits = plsc.bitcast(v, jnp.int32)
    rb = bits + jnp.int32(0x7FFF) + ((bits >> 16) & jnp.int32(1))
    return plsc.bitcast(rb & jnp.int32(-65536), jnp.float32)


def _rsqrt3(v):
    bits = plsc.bitcast(v, jnp.int32)
    y = plsc.bitcast(jnp.int32(0x5F3759DF) - (bits >> 1), jnp.float32)
    for _ in range(3):
        y = y * (1.5 - 0.5 * v * y * y)
    return y


def _sc_body(x_hbm, gkt_hbm, dbase_hbm, out_hbm,
             xs, stage, hist, vsrow, vdrow, gkb, dbase, outb):
    wid = lax.axis_index("s") * 2 + lax.axis_index("c")
    pltpu.sync_copy(gkt_hbm, gkb)
    pltpu.sync_copy(dbase_hbm, dbase)
    iot = lax.iota(jnp.int32, 16)
    zvec = jnp.zeros((16,), jnp.float32)

    for img in range(IMGS_PER_TILE):
        b = wid * IMGS_PER_TILE + img

        def band1(h, _):
            r0 = h * 8
            for c in range(3):
                pltpu.sync_copy(x_hbm.at[b, c, pl.ds(r0, 8)], stage.at[c])

            def row1(rr, _):
                def vec1(j, _):
                    off = pl.ds(j * 16, 16)
                    v = (_bf16_round(stage[0, rr, off])
                         + _bf16_round(stage[1, rr, off])
                         + _bf16_round(stage[2, rr, off]))
                    xs[r0 + rr, off] = v
                    return 0
                lax.fori_loop(0, 14, vec1, 0, unroll=2)
                return 0
            lax.fori_loop(0, 8, row1, 0)
            return 0
        lax.fori_loop(0, 28, band1, 0)

        def zh(i, _):
            hist[pl.ds(i * 16, 16)] = zvec
            return 0
        lax.fori_loop(0, FEAT // 16, zh, 0)

        def row2(r, _):
            rm = jnp.where(r == 0, 1, r - 1)
            rp = jnp.where(r == H - 1, H - 2, r + 1)
            grow = r & 15
            base = (r >> 3) * WP

            def vec2a(j, _):
                off = pl.ds(j * 16, 16)
                a = xs[rm, off]
                c0 = xs[r, off]
                bb = xs[rp, off]
                vsrow[off] = a + 2.0 * c0 + bb
                vdrow[off] = a - bb
                return 0
            lax.fori_loop(0, 14, vec2a, 0, unroll=2)

            def vec2b(j, _):
                w0 = j * 16
                off = pl.ds(w0, 16)
                wl = w0 - 1 + iot
                wl = jnp.where(wl < 0, 1, wl)
                wr = w0 + 1 + iot
                wr = jnp.where(wr > W - 1, W - 2, wr)
                gx = plsc.load_gather(vsrow, [wl]) - plsc.load_gather(vsrow, [wr])
                gy = (plsc.load_gather(vdrow, [wl]) + 2.0 * vdrow[off]
                      + plsc.load_gather(vdrow, [wr]))
                g2 = gx * gx + gy * gy
                wn = g2 * _rsqrt3(g2) * gkb[grow, off]
                flip = (gx < 0.0) | ((gx == 0.0) & (gy < 0.0))
                fx = jnp.where(flip, -gx, gx)
                fy = jnp.where(flip, -gy, gy)
                cnt = jnp.zeros((16,), jnp.int32)
                one = jnp.ones((16,), jnp.int32)
                czero = jnp.zeros((16,), jnp.int32)
                for k in range(NBINS - 1):
                    t = fx * _COS[k] - fy * _SIN[k]
                    cnt = cnt + jnp.where(t >= 0.0, one, czero)
                cell = base + ((w0 + iot) >> 3)
                plsc.addupdate_scatter(hist, [cnt * NCELL + cell], wn)
                return 0
            lax.fori_loop(0, 14, vec2b, 0)
            return 0
        lax.fori_loop(0, H, row2, 0)

        def cellvec(v, _):
            off = pl.ds(v * 16, 16)
            hk = [hist[pl.ds(k * NCELL + v * 16, 16)] for k in range(NBINS)]
            s = zvec
            for k in range(NBINS):
                s = s + hk[k] * hk[k]
            y = _rsqrt3(s)
            db = dbase[off]
            for k in range(NBINS):
                plsc.store_scatter(outb, [db + 4 * k], hk[k] * y)
            return 0
        lax.fori_loop(0, NCELL // 16, cellvec, 0)

        pltpu.sync_copy(outb, out_hbm.at[b])


def kernel(x, weight_x, weight_y, gaussian_kernel):
    b = x.shape[0]
    gkt = jnp.tile(gaussian_kernel, (1, W // 16))

    hc, wc = np.meshgrid(np.arange(HP), np.arange(WP), indexing="ij")
    dest = ((hc >> 1) * 14 + (wc >> 1)) * 36 + (hc & 1) * 2 + (wc & 1)
    dbase = jnp.asarray(dest.reshape(-1).astype(np.int32))

    mesh = plsc.VectorSubcoreMesh(core_axis_name="c", subcore_axis_name="s")
    run = functools.partial(
        pl.kernel, mesh=mesh,
        out_type=jax.ShapeDtypeStruct((b, FEAT), jnp.float32),
        scratch_types=[
            pltpu.VMEM((H, W), jnp.float32),
            pltpu.VMEM((3, 8, W), jnp.float32),
            pltpu.VMEM((FEAT,), jnp.float32),
            pltpu.VMEM((W,), jnp.float32),
            pltpu.VMEM((W,), jnp.float32),
            pltpu.VMEM((16, W), jnp.float32),
            pltpu.VMEM((NCELL,), jnp.int32),
            pltpu.VMEM((FEAT,), jnp.float32),
        ],
        compiler_params=pltpu.CompilerParams(needs_layout_passes=False),
    )(_sc_body)

    feat = run(x, gkt, dbase)
    return feat.reshape(b, 196, 36)


# speedup vs baseline: 1.0309x; 1.0309x over previous
import functools
import math

import jax
import jax.numpy as jnp
import numpy as np
from jax import lax
from jax.experimental import pallas as pl
from jax.experimental.pallas import tpu as pltpu, tpu_sc as plsc

NBINS = 9
H = W = 224
HP = WP = 28
NCELL = HP * WP
FEAT = NCELL * NBINS
NTILES = 32
IMGS_PER_TILE = 64 // NTILES

_COS = [math.cos(k * math.pi / NBINS) for k in range(1, NBINS)]
_SIN = [math.sin(k * math.pi / NBINS) for k in range(1, NBINS)]


def _bf16_round(v):
    bits = plsc.bitcast(v, jnp.int32)
    rb = bits + jnp.int32(0x7FFF) + ((bits >> 16) & jnp.int32(1))
    return plsc.bitcast(rb & jnp.int32(-65536), jnp.float32)


def _rsqrt2(v):
    bits = plsc.bitcast(v, jnp.int32)
    y = plsc.bitcast(jnp.int32(0x5F3759DF) - (bits >> 1), jnp.float32)
    for _ in range(2):
        y = y * (1.5 - 0.5 * v * y * y)
    return y


def _sc_body(x_hbm, gkt_hbm, dbase_hbm, out_hbm,
             xs, stage, hist, vsrow, vdrow, gkb, dbase, outb):
    wid = lax.axis_index("s") * 2 + lax.axis_index("c")
    pltpu.sync_copy(gkt_hbm, gkb)
    pltpu.sync_copy(dbase_hbm, dbase)
    iot = lax.iota(jnp.int32, 16)
    zvec = jnp.zeros((16,), jnp.float32)

    for img in range(IMGS_PER_TILE):
        b = wid * IMGS_PER_TILE + img

        def band1(h, _):
            r0 = h * 8
            for c in range(3):
                pltpu.sync_copy(x_hbm.at[b, c, pl.ds(r0, 8)], stage.at[c])

            def row1(rr, _):
                def vec1(j, _):
                    off = pl.ds(j * 16, 16)
                    v = (_bf16_round(stage[0, rr, off])
                         + _bf16_round(stage[1, rr, off])
                         + _bf16_round(stage[2, rr, off]))
                    xs[r0 + rr, off] = v
                    return 0
                lax.fori_loop(0, 14, vec1, 0, unroll=2)
                return 0
            lax.fori_loop(0, 8, row1, 0)
            return 0
        lax.fori_loop(0, 28, band1, 0)

        def zh(i, _):
            hist[pl.ds(i * 16, 16)] = zvec
            return 0
        lax.fori_loop(0, FEAT // 16, zh, 0)

        edge_src = jnp.where(iot == 0, 2, W - 1)
        edge_dst = jnp.where(iot == 0, 0, W + 1)
        edge_msk = iot < 2
        one = jnp.ones((16,), jnp.int32)
        czero = jnp.zeros((16,), jnp.int32)

        def row2(r, _):
            rm = jnp.where(r == 0, 1, r - 1)
            rp = jnp.where(r == H - 1, H - 2, r + 1)
            grow = r & 15
            base = (r >> 3) * WP

            def vec2a(j, _):
                off = pl.ds(j * 16, 16)
                offp = pl.ds(j * 16 + 1, 16)
                a = xs[rm, off]
                c0 = xs[r, off]
                bb = xs[rp, off]
                vsrow[offp] = a + 2.0 * c0 + bb
                vdrow[offp] = a - bb
                return 0
            lax.fori_loop(0, 14, vec2a, 0, unroll=2)
            plsc.store_scatter(vsrow, [edge_dst],
                               plsc.load_gather(vsrow, [edge_src]), mask=edge_msk)
            plsc.store_scatter(vdrow, [edge_dst],
                               plsc.load_gather(vdrow, [edge_src]), mask=edge_msk)

            def vec2b(j, _):
                w0 = j * 16
                gx = vsrow[pl.ds(w0, 16)] - vsrow[pl.ds(w0 + 2, 16)]
                gy = (vdrow[pl.ds(w0, 16)] + 2.0 * vdrow[pl.ds(w0 + 1, 16)]
                      + vdrow[pl.ds(w0 + 2, 16)])
                g2 = gx * gx + gy * gy
                wn = g2 * _rsqrt2(g2) * gkb[grow, pl.ds(w0, 16)]
                flip = (gx < 0.0) | ((gx == 0.0) & (gy < 0.0))
                fx = jnp.where(flip, -gx, gx)
                fy = jnp.where(flip, -gy, gy)
                cnt = jnp.zeros((16,), jnp.int32)
                for k in range(4):
                    p = fx * _COS[k]
                    q = fy * _SIN[k]
                    cnt = cnt + jnp.where(p - q >= 0.0, one, czero)
                    cnt = cnt + jnp.where(p + q <= 0.0, one, czero)
                cell = base + ((w0 + iot) >> 3)
                plsc.addupdate_scatter(hist, [cnt * NCELL + cell], wn)
                return 0
            lax.fori_loop(0, 14, vec2b, 0, unroll=2)
            return 0
        lax.fori_loop(0, H, row2, 0)

        def cellvec(v, _):
            off = pl.ds(v * 16, 16)
            hk = [hist[pl.ds(k * NCELL + v * 16, 16)] for k in range(NBINS)]
            s = zvec
            for k in range(NBINS):
                s = s + hk[k] * hk[k]
            y = _rsqrt2(s)
            db = dbase[off]
            for k in range(NBINS):
                plsc.store_scatter(outb, [db + 4 * k], hk[k] * y)
            return 0
        lax.fori_loop(0, NCELL // 16, cellvec, 0)

        pltpu.sync_copy(outb, out_hbm.at[b])


def kernel(x, weight_x, weight_y, gaussian_kernel):
    b = x.shape[0]
    gkt = jnp.tile(gaussian_kernel, (1, W // 16))

    hc, wc = np.meshgrid(np.arange(HP), np.arange(WP), indexing="ij")
    dest = ((hc >> 1) * 14 + (wc >> 1)) * 36 + (hc & 1) * 2 + (wc & 1)
    dbase = jnp.asarray(dest.reshape(-1).astype(np.int32))

    mesh = plsc.VectorSubcoreMesh(core_axis_name="c", subcore_axis_name="s")
    run = functools.partial(
        pl.kernel, mesh=mesh,
        out_type=jax.ShapeDtypeStruct((b, FEAT), jnp.float32),
        scratch_types=[
            pltpu.VMEM((H, W), jnp.float32),
            pltpu.VMEM((3, 8, W), jnp.float32),
            pltpu.VMEM((FEAT,), jnp.float32),
            pltpu.VMEM((W + 2,), jnp.float32),
            pltpu.VMEM((W + 2,), jnp.float32),
            pltpu.VMEM((16, W), jnp.float32),
            pltpu.VMEM((NCELL,), jnp.int32),
            pltpu.VMEM((FEAT,), jnp.float32),
        ],
        compiler_params=pltpu.CompilerParams(needs_layout_passes=False),
    )(_sc_body)

    feat = run(x, gkt, dbase)
    return feat.reshape(b, 196, 36)


# speedup vs baseline: 1.3723x; 1.3311x over previous
import functools
import math

import jax
import jax.numpy as jnp
import numpy as np
from jax import lax
from jax.experimental import pallas as pl
from jax.experimental.pallas import tpu as pltpu, tpu_sc as plsc

NBINS = 9
H = W = 224
HP = WP = 28
NCELL = HP * WP
FEAT = NCELL * NBINS
NTILES = 32
IMGS_PER_TILE = 64 // NTILES

_COS = [math.cos(k * math.pi / NBINS) for k in range(1, NBINS)]
_SIN = [math.sin(k * math.pi / NBINS) for k in range(1, NBINS)]


def _bf16_round(v):
    bits = plsc.bitcast(v, jnp.int32)
    rb = bits + jnp.int32(0x7FFF) + ((bits >> 16) & jnp.int32(1))
    return plsc.bitcast(rb & jnp.int32(-65536), jnp.float32)


def _rsqrt2(v):
    bits = plsc.bitcast(v, jnp.int32)
    y = plsc.bitcast(jnp.int32(0x5F3759DF) - (bits >> 1), jnp.float32)
    for _ in range(2):
        y = y * (1.5 - 0.5 * v * y * y)
    return y


def _sc_body(x_hbm, gkt_hbm, dbase_hbm, out_hbm,
             xs, stage, hist, vsrow, vdrow, gkb, dbase, outb):
    wid = lax.axis_index("s") * 2 + lax.axis_index("c")
    pltpu.sync_copy(gkt_hbm, gkb)
    pltpu.sync_copy(dbase_hbm, dbase)
    iot = lax.iota(jnp.int32, 16)
    zvec = jnp.zeros((16,), jnp.float32)

    for img in range(IMGS_PER_TILE):
        b = wid * IMGS_PER_TILE + img

        pltpu.sync_copy(x_hbm.at[b, 0], xs)
        for c in range(1, 3):
            for hh in range(2):
                pltpu.sync_copy(x_hbm.at[b, c, pl.ds(hh * 112, 112)], stage)

                def row1(rr, _):
                    r = hh * 112 + rr

                    def vec1(j, _):
                        off = pl.ds(j * 16, 16)
                        prev = xs[r, off]
                        if c == 1:
                            prev = _bf16_round(prev)
                        xs[r, off] = prev + _bf16_round(stage[rr, off])
                        return 0
                    lax.fori_loop(0, 14, vec1, 0, unroll=2)
                    return 0
                lax.fori_loop(0, 112, row1, 0)

        def zh(i, _):
            hist[pl.ds(i * 16, 16)] = zvec
            return 0
        lax.fori_loop(0, FEAT // 16, zh, 0)

        edge_src = jnp.where(iot == 0, 2, W - 1)
        edge_dst = jnp.where(iot == 0, 0, W + 1)
        edge_msk = iot < 2
        one = jnp.ones((16,), jnp.int32)
        czero = jnp.zeros((16,), jnp.int32)

        def row2(r, _):
            rm = jnp.where(r == 0, 1, r - 1)
            rp = jnp.where(r == H - 1, H - 2, r + 1)
            grow = r & 15
            base = (r >> 3) * WP

            def vec2a(j, _):
                off = pl.ds(j * 16, 16)
                offp = pl.ds(j * 16 + 1, 16)
                a = xs[rm, off]
                c0 = xs[r, off]
                bb = xs[rp, off]
                vsrow[offp] = a + 2.0 * c0 + bb
                vdrow[offp] = a - bb
                return 0
            lax.fori_loop(0, 14, vec2a, 0, unroll=2)
            plsc.store_scatter(vsrow, [edge_dst],
                               plsc.load_gather(vsrow, [edge_src]), mask=edge_msk)
            plsc.store_scatter(vdrow, [edge_dst],
                               plsc.load_gather(vdrow, [edge_src]), mask=edge_msk)

            def vec2b(j, _):
                w0 = j * 16
                gx = vsrow[pl.ds(w0, 16)] - vsrow[pl.ds(w0 + 2, 16)]
                gy = (vdrow[pl.ds(w0, 16)] + 2.0 * vdrow[pl.ds(w0 + 1, 16)]
                      + vdrow[pl.ds(w0 + 2, 16)])
                g2 = gx * gx + gy * gy
                wn = g2 * _rsqrt2(g2) * gkb[grow, pl.ds(w0, 16)]
                flip = (gx < 0.0) | ((gx == 0.0) & (gy < 0.0))
                fx = jnp.where(flip, -gx, gx)
                fy = jnp.where(flip, -gy, gy)
                cnt = jnp.zeros((16,), jnp.int32)
                for k in range(4):
                    p = fx * _COS[k]
                    q = fy * _SIN[k]
                    cnt = cnt + jnp.where(p - q >= 0.0, one, czero)
                    cnt = cnt + jnp.where(p + q <= 0.0, one, czero)
                cell = base + ((w0 + iot) >> 3)
                plsc.addupdate_scatter(hist, [cnt * NCELL + cell], wn)
                return 0
            lax.fori_loop(0, 14, vec2b, 0, unroll=2)
            return 0
        lax.fori_loop(0, H, row2, 0)

        def cellvec(v, _):
            off = pl.ds(v * 16, 16)
            hk = [hist[pl.ds(k * NCELL + v * 16, 16)] for k in range(NBINS)]
            s = zvec
            for k in range(NBINS):
                s = s + hk[k] * hk[k]
            y = _rsqrt2(s)
            db = dbase[off]
            for k in range(NBINS):
                plsc.store_scatter(outb, [db + 4 * k], hk[k] * y)
            return 0
        lax.fori_loop(0, NCELL // 16, cellvec, 0)

        pltpu.sync_copy(outb, out_hbm.at[b])


def kernel(x, weight_x, weight_y, gaussian_kernel):
    b = x.shape[0]
    gkt = jnp.tile(gaussian_kernel, (1, W // 16))

    hc, wc = np.meshgrid(np.arange(HP), np.arange(WP), indexing="ij")
    dest = ((hc >> 1) * 14 + (wc >> 1)) * 36 + (hc & 1) * 2 + (wc & 1)
    dbase = jnp.asarray(dest.reshape(-1).astype(np.int32))

    mesh = plsc.VectorSubcoreMesh(core_axis_name="c", subcore_axis_name="s")
    run = functools.partial(
        pl.kernel, mesh=mesh,
        out_type=jax.ShapeDtypeStruct((b, FEAT), jnp.float32),
        scratch_types=[
            pltpu.VMEM((H, W), jnp.float32),
            pltpu.VMEM((112, W), jnp.float32),
            pltpu.VMEM((FEAT,), jnp.float32),
            pltpu.VMEM((W + 2,), jnp.float32),
            pltpu.VMEM((W + 2,), jnp.float32),
            pltpu.VMEM((16, W), jnp.float32),
            pltpu.VMEM((NCELL,), jnp.int32),
            pltpu.VMEM((FEAT,), jnp.float32),
        ],
        compiler_params=pltpu.CompilerParams(needs_layout_passes=False),
    )(_sc_body)

    feat = run(x, gkt, dbase)
    return feat.reshape(b, 196, 36)


# speedup vs baseline: 2.3872x; 1.7396x over previous
import functools
import math

import jax
import jax.numpy as jnp
import numpy as np
from jax import lax
from jax.experimental import pallas as pl
from jax.experimental.pallas import tpu as pltpu, tpu_sc as plsc

NBINS = 9
POOL = 8
GW = 16
H = W = 224
HP = WP = H // POOL
NCELL = HP * WP
FEAT = NCELL * NBINS
NTILES = 32
SC_IMGS = 32

_COS = [math.cos(k * math.pi / NBINS) for k in range(1, NBINS)]
_SIN = [math.sin(k * math.pi / NBINS) for k in range(1, NBINS)]


def _bf16_round(v):
    bits = plsc.bitcast(v, jnp.int32)
    rb = bits + jnp.int32(0x7FFF) + ((bits >> 16) & jnp.int32(1))
    return plsc.bitcast(rb & jnp.int32(-65536), jnp.float32)


def _rsqrt2(v):
    bits = plsc.bitcast(v, jnp.int32)
    y = plsc.bitcast(jnp.int32(0x5F3759DF) - (bits >> 1), jnp.float32)
    for _ in range(2):
        y = y * (1.5 - 0.5 * v * y * y)
    return y


def _sc_body(x_hbm, gkt_hbm, dbase_hbm, out_hbm,
             xs, stage, hist, vsrow, vdrow, gkb, dbase, outb):
    wid = lax.axis_index("s") * 2 + lax.axis_index("c")
    pltpu.sync_copy(gkt_hbm, gkb)
    pltpu.sync_copy(dbase_hbm, dbase)
    iot = lax.iota(jnp.int32, 16)
    zvec = jnp.zeros((16,), jnp.float32)

    b = (64 - SC_IMGS) + wid

    pltpu.sync_copy(x_hbm.at[b, 0], xs)
    for c in range(1, 3):
        for hh in range(2):
            pltpu.sync_copy(x_hbm.at[b, c, pl.ds(hh * 112, 112)], stage)

            def row1(rr, _):
                r = hh * 112 + rr

                def vec1(j, _):
                    off = pl.ds(j * 16, 16)
                    prev = xs[r, off]
                    if c == 1:
                        prev = _bf16_round(prev)
                    xs[r, off] = prev + _bf16_round(stage[rr, off])
                    return 0
                lax.fori_loop(0, 14, vec1, 0, unroll=2)
                return 0
            lax.fori_loop(0, 112, row1, 0)

    def zh(i, _):
        hist[pl.ds(i * 16, 16)] = zvec
        return 0
    lax.fori_loop(0, FEAT // 16, zh, 0)

    edge_src = jnp.where(iot == 0, 2, W - 1)
    edge_dst = jnp.where(iot == 0, 0, W + 1)
    edge_msk = iot < 2
    one = jnp.ones((16,), jnp.int32)
    czero = jnp.zeros((16,), jnp.int32)

    def row2(r, _):
        rm = jnp.where(r == 0, 1, r - 1)
        rp = jnp.where(r == H - 1, H - 2, r + 1)
        grow = r & 15
        base = (r >> 3) * WP

        def vec2a(j, _):
            off = pl.ds(j * 16, 16)
            offp = pl.ds(j * 16 + 1, 16)
            a = xs[rm, off]
            c0 = xs[r, off]
            bb = xs[rp, off]
            vsrow[offp] = a + 2.0 * c0 + bb
            vdrow[offp] = a - bb
            return 0
        lax.fori_loop(0, 14, vec2a, 0, unroll=2)
        plsc.store_scatter(vsrow, [edge_dst],
                           plsc.load_gather(vsrow, [edge_src]), mask=edge_msk)
        plsc.store_scatter(vdrow, [edge_dst],
                           plsc.load_gather(vdrow, [edge_src]), mask=edge_msk)

        def vec2b(j, _):
            w0 = j * 16
            gx = vsrow[pl.ds(w0, 16)] - vsrow[pl.ds(w0 + 2, 16)]
            gy = (vdrow[pl.ds(w0, 16)] + 2.0 * vdrow[pl.ds(w0 + 1, 16)]
                  + vdrow[pl.ds(w0 + 2, 16)])
            g2 = gx * gx + gy * gy
            wn = g2 * _rsqrt2(g2) * gkb[grow, pl.ds(w0, 16)]
            flip = (gx < 0.0) | ((gx == 0.0) & (gy < 0.0))
            fx = jnp.where(flip, -gx, gx)
            fy = jnp.where(flip, -gy, gy)
            cnt = jnp.zeros((16,), jnp.int32)
            for k in range(4):
                p = fx * _COS[k]
                q = fy * _SIN[k]
                cnt = cnt + jnp.where(p - q >= 0.0, one, czero)
                cnt = cnt + jnp.where(p + q <= 0.0, one, czero)
            cell = base + ((w0 + iot) >> 3)
            plsc.addupdate_scatter(hist, [cnt * NCELL + cell], wn)
            return 0
        lax.fori_loop(0, 14, vec2b, 0, unroll=2)
        return 0
    lax.fori_loop(0, H, row2, 0)

    def cellvec(v, _):
        off = pl.ds(v * 16, 16)
        hk = [hist[pl.ds(k * NCELL + v * 16, 16)] for k in range(NBINS)]
        s = zvec
        for k in range(NBINS):
            s = s + hk[k] * hk[k]
        y = _rsqrt2(s)
        db = dbase[off]
        for k in range(NBINS):
            plsc.store_scatter(outb, [db + 4 * k], hk[k] * y)
        return 0
    lax.fori_loop(0, NCELL // 16, cellvec, 0)

    pltpu.sync_copy(outb, out_hbm.at[wid])


def _tc_body(x_ref, gk_ref, pool_ref, out_ref):
    xb0 = x_ref[0, 0].astype(jnp.bfloat16).astype(jnp.float32)
    xb1 = x_ref[0, 1].astype(jnp.bfloat16).astype(jnp.float32)
    xb2 = x_ref[0, 2].astype(jnp.bfloat16).astype(jnp.float32)
    xs = xb0 + xb1 + xb2

    xp = jnp.concatenate([xs[1:2], xs, xs[H - 2:H - 1]], axis=0)
    xp = jnp.concatenate([xp[:, 1:2], xp, xp[:, W - 2:W - 1]], axis=1)

    vs = xp[:-2] + 2.0 * xp[1:-1] + xp[2:]
    vd = xp[:-2] - xp[2:]
    gx = vs[:, :-2] - vs[:, 2:]
    gy = vd[:, :-2] + 2.0 * vd[:, 1:-1] + vd[:, 2:]

    norm = jnp.sqrt(gx * gx + gy * gy) * gk_ref[...]

    flip = (gx < 0.0) | ((gx == 0.0) & (gy < 0.0))
    fx = jnp.where(flip, -gx, gx)
    fy = jnp.where(flip, -gy, gy)
    binv = jnp.zeros(fx.shape, dtype=jnp.int32)
    for k in range(1, NBINS):
        beta = k * math.pi / NBINS
        t = fx * math.cos(beta) - fy * math.sin(beta)
        binv = binv + jnp.where(t >= 0.0, 1, 0).astype(jnp.int32)

    p = pool_ref[...]
    hist = []
    for k in range(NBINS):
        m = jnp.where(binv == k, norm, 0.0)
        pm = jax.lax.dot_general(p, m, (((1,), (0,)), ((), ())),
                                 preferred_element_type=jnp.float32,
                                 precision=jax.lax.Precision.HIGHEST)
        hist.append(jax.lax.dot_general(pm, p, (((1,), (1,)), ((), ())),
                                        preferred_element_type=jnp.float32,
                                        precision=jax.lax.Precision.HIGHEST))
    h3 = jnp.stack(hist, axis=0)

    n2 = jnp.sqrt(jnp.sum(h3 * h3, axis=0, keepdims=True))
    out_ref[0] = h3 / jnp.maximum(n2, 1e-12)


def kernel(x, weight_x, weight_y, gaussian_kernel):
    b = x.shape[0]
    tc_n = b - SC_IMGS
    rep = H // GW

    gk224 = jnp.tile(gaussian_kernel, (rep, rep))
    pool_mat = jnp.asarray(
        np.repeat(np.eye(HP, dtype=np.float32), POOL, axis=1))
    normed = pl.pallas_call(
        _tc_body,
        grid=(tc_n,),
        in_specs=[
            pl.BlockSpec((1, 3, H, W), lambda i: (i, 0, 0, 0)),
            pl.BlockSpec((H, W), lambda i: (0, 0)),
            pl.BlockSpec((HP, H), lambda i: (0, 0)),
        ],
        out_specs=pl.BlockSpec((1, NBINS, HP, WP), lambda i: (i, 0, 0, 0)),
        out_shape=jax.ShapeDtypeStruct((tc_n, NBINS, HP, WP), jnp.float32),
    )(x, gk224, pool_mat)
    feat_tc = normed.transpose(0, 2, 3, 1)
    feat_tc = feat_tc.reshape(tc_n, 14, 2, 14, 2, NBINS)
    feat_tc = feat_tc.transpose(0, 1, 3, 5, 2, 4)
    feat_tc = feat_tc.reshape(tc_n, 196, 36)

    gkt = jnp.tile(gaussian_kernel, (1, W // 16))
    hc, wc = np.meshgrid(np.arange(HP), np.arange(WP), indexing="ij")
    dest = ((hc >> 1) * 14 + (wc >> 1)) * 36 + (hc & 1) * 2 + (wc & 1)
    dbase = jnp.asarray(dest.reshape(-1).astype(np.int32))

    mesh = plsc.VectorSubcoreMesh(core_axis_name="c", subcore_axis_name="s")
    sc_run = functools.partial(
        pl.kernel, mesh=mesh,
        out_type=jax.ShapeDtypeStruct((SC_IMGS, FEAT), jnp.float32),
        scratch_types=[
            pltpu.VMEM((H, W), jnp.float32),
            pltpu.VMEM((112, W), jnp.float32),
            pltpu.VMEM((FEAT,), jnp.float32),
            pltpu.VMEM((W + 2,), jnp.float32),
            pltpu.VMEM((W + 2,), jnp.float32),
            pltpu.VMEM((16, W), jnp.float32),
            pltpu.VMEM((NCELL,), jnp.int32),
            pltpu.VMEM((FEAT,), jnp.float32),
        ],
        compiler_params=pltpu.CompilerParams(needs_layout_passes=False),
    )(_sc_body)
    feat_sc = sc_run(x, gkt, dbase).reshape(SC_IMGS, 196, 36)

    return jnp.concatenate([feat_tc, feat_sc], axis=0)
